# baseline (speedup 1.0000x reference)
import functools

import jax
import jax.numpy as jnp
from jax import lax
from jax.experimental import pallas as pl
from jax.experimental.pallas import tpu as pltpu
from jax.experimental.pallas import tpu_sc as plsc

VOCAB = 100000
EMB = 128
BATCH = 4096
CTX = 50
N_NEGS = 20
PAD = 0
SEED = 42

NSAMP = BATCH * N_NEGS
ROWS = 784
VPAD = ROWS * 128
NW = 32
SAMP_PER_W = NSAMP // NW
B_PER_W = BATCH // NW
SEARCH_STEPS = 17
NPAD = 32
EX_PER_CHUNK = 4
NCHUNK = B_PER_W // EX_PER_CHUNK
JC = EMB // 16


def _cdf_rng_body(w_ref, cdf_ref, u_ref):
    w = w_ref[...]
    k = lax.broadcasted_iota(jnp.int32, (128, 128), 0)
    j = lax.broadcasted_iota(jnp.int32, (128, 128), 1)
    tri = (k <= j).astype(jnp.float32)
    rowcum = jnp.dot(w, tri, preferred_element_type=jnp.float32)
    totals = rowcum[:, 127:128]
    r1 = lax.broadcasted_iota(jnp.int32, (ROWS, ROWS), 0)
    r2 = lax.broadcasted_iota(jnp.int32, (ROWS, ROWS), 1)
    ls = (r2 < r1).astype(jnp.float32)
    prefix = jnp.dot(ls, totals, preferred_element_type=jnp.float32)
    cdf = rowcum + prefix
    cdf_ref[...] = cdf
    total = cdf[ROWS - 1:ROWS, 127:128]

    pltpu.prng_seed(SEED)
    bits = pltpu.prng_random_bits((NSAMP // 128, 128))
    bits = lax.bitcast_convert_type(bits, jnp.uint32)
    mant = lax.shift_right_logical(bits, jnp.uint32(9)) | jnp.uint32(0x3F800000)
    unif = lax.bitcast_convert_type(mant, jnp.float32) - 1.0
    u_ref[...] = unif * total


def _cdf_and_uniforms(weights):
    w2d = jnp.pad(weights, (0, VPAD - VOCAB)).reshape(ROWS, 128)
    cdf, u = pl.pallas_call(
        _cdf_rng_body,
        out_shape=(
            jax.ShapeDtypeStruct((ROWS, 128), jnp.float32),
            jax.ShapeDtypeStruct((NSAMP // 128, 128), jnp.float32),
        ),
    )(w2d)
    return cdf.reshape(VPAD), u.reshape(NSAMP)


def _sample_body(cdf_hbm, u_hbm, nw_hbm, cdf_v, u_v, out_v, sem):
    wid = lax.axis_index("s") * 2 + lax.axis_index("c")
    base = wid * SAMP_PER_W
    pltpu.async_copy(cdf_hbm, cdf_v, sem).wait()
    pltpu.sync_copy(u_hbm.at[pl.ds(base, SAMP_PER_W)], u_v)

    nlanes = 8

    def group(g, carry):
        ums = [u_v[pl.ds((nlanes * g + t) * 16, 16)] for t in range(nlanes)]
        lo0 = [jnp.zeros((16,), jnp.int32)] * nlanes
        hi0 = [jnp.full((16,), VOCAB - 1, jnp.int32)] * nlanes

        def step(_, lohi):
            los, his = lohi
            nlos, nhis = [], []
            for t in range(nlanes):
                mid = lax.shift_right_arithmetic(los[t] + his[t], 1)
                c = plsc.load_gather(cdf_v, [mid])
                gt = c > ums[t]
                nlos.append(jnp.where(gt, los[t], mid + 1))
                nhis.append(jnp.where(gt, mid, his[t]))
            return (nlos, nhis)

        _, his = lax.fori_loop(0, SEARCH_STEPS, step, (lo0, hi0))
        for t in range(nlanes):
            out_v[pl.ds((nlanes * g + t) * 16, 16)] = his[t]
        return carry

    lax.fori_loop(0, SAMP_PER_W // (16 * nlanes), group, 0)
    pltpu.sync_copy(out_v, nw_hbm.at[pl.ds(base, SAMP_PER_W)])


def _sample_negatives(cdf_flat, u_flat):
    mesh = plsc.VectorSubcoreMesh(core_axis_name="c", subcore_axis_name="s")
    f = functools.partial(
        pl.kernel,
        out_type=jax.ShapeDtypeStruct((NSAMP,), jnp.int32),
        mesh=mesh,
        scratch_types=[
            pltpu.VMEM((VPAD,), jnp.float32),
            pltpu.VMEM((SAMP_PER_W,), jnp.float32),
            pltpu.VMEM((SAMP_PER_W,), jnp.int32),
            pltpu.SemaphoreType.DMA,
        ],
        compiler_params=pltpu.CompilerParams(needs_layout_passes=False),
    )(_sample_body)
    return f(cdf_flat, u_flat)


def _gather_body(isf_hbm, mw_hbm, nw_hbm, tab_hbm, o_hbm, n_hbm,
                 sidx_v, rows0, rows1, embsum_v, midx_v, mrow_v,
                 negidx_v, tmp_v, n_out_v, o_out_v, sem0, sem1, sem2):
    sems = (sem0, sem1)
    rows_b = (rows0, rows1)
    wid = lax.axis_index("s") * 2 + lax.axis_index("c")
    b0 = wid * B_PER_W
    s0 = wid * SAMP_PER_W
    iota16 = lax.iota(jnp.int32, 16)

    pltpu.sync_copy(mw_hbm.at[pl.ds(b0, B_PER_W)], midx_v)
    pltpu.async_copy(tab_hbm.at[midx_v], mrow_v, sem2)

    pltpu.sync_copy(isf_hbm.at[pl.ds(b0 * CTX, B_PER_W * CTX)], sidx_v)

    def _ctx_start(c, b):
        pltpu.async_copy(
            tab_hbm.at[sidx_v.at[pl.ds(c * EX_PER_CHUNK * CTX,
                                       EX_PER_CHUNK * CTX)]],
            rows_b[b], sems[b])

    def _ctx_wait(c, b):
        pltpu.make_async_copy(
            tab_hbm.at[sidx_v.at[pl.ds(c * EX_PER_CHUNK * CTX,
                                       EX_PER_CHUNK * CTX)]],
            rows_b[b], sems[b]).wait()

    _ctx_start(0, 0)
    _ctx_start(1, 1)

    def ctx_pair(p, carry):
        for b in range(2):
            c = 2 * p + b
            _ctx_wait(c, b)
            for t in range(EX_PER_CHUNK):
                def rblk(rb, accs):
                    out = list(accs)
                    for rr in range(10):
                        r = t * CTX + rb * 10 + rr
                        for j in range(JC):
                            out[j] = out[j] + rows_b[b][r, pl.ds(j * 16, 16)]
                    return tuple(out)

                accs = lax.fori_loop(
                    0, CTX // 10, rblk,
                    tuple(jnp.zeros((16,), jnp.float32) for _ in range(JC)))
                ex = c * EX_PER_CHUNK + t
                for j in range(JC):
                    embsum_v[ex, pl.ds(j * 16, 16)] = accs[j]

            @pl.when(p < NCHUNK // 2 - 1)
            def _():
                _ctx_start(c + 2, b)
        return carry

    lax.fori_loop(0, NCHUNK // 2, ctx_pair, 0)

    pltpu.make_async_copy(tab_hbm.at[midx_v], mrow_v, sem2).wait()

    def ogroup(g, carry):
        for e in range(16):
            ex = g * 16 + e
            acc = jnp.zeros((16,), jnp.float32)
            for j in range(JC):
                acc = acc + (mrow_v[ex, pl.ds(j * 16, 16)] *
                             embsum_v[ex, pl.ds(j * 16, 16)])
            tmp_v[e, :] = acc
        tot = jnp.zeros((16,), jnp.float32)
        for j in range(16):
            tot = tot + plsc.load_gather(
                tmp_v, [iota16, jnp.full((16,), j, jnp.int32)])
        o_out_v[pl.ds(g * 16, 16)] = tot
        return carry

    lax.fori_loop(0, B_PER_W // 16, ogroup, 0)
    pltpu.sync_copy(o_out_v, o_hbm.at[pl.ds(b0, B_PER_W)])

    pltpu.sync_copy(nw_hbm.at[pl.ds(s0, SAMP_PER_W)], negidx_v)

    def _neg_start(c, b):
        pltpu.async_copy(
            tab_hbm.at[negidx_v.at[pl.ds(c * 80, 80)]], rows_b[b].at[:80],
            sems[b])

    def _neg_wait(c, b):
        pltpu.make_async_copy(
            tab_hbm.at[negidx_v.at[pl.ds(c * 80, 80)]], rows_b[b].at[:80],
            sems[b]).wait()

    _neg_start(0, 0)
    _neg_start(1, 1)

    def neg_pair(p, carry):
        for b in range(2):
            c = 2 * p + b
            _neg_wait(c, b)
            for t in range(EX_PER_CHUNK):
                ex = c * EX_PER_CHUNK + t
                embregs = [embsum_v[ex, pl.ds(j * 16, 16)] for j in range(JC)]
                for kk in range(N_NEGS):
                    acc = jnp.zeros((16,), jnp.float32)
                    for j in range(JC):
                        acc = acc + (rows_b[b][t * N_NEGS + kk,
                                               pl.ds(j * 16, 16)] *
                                     embregs[j])
                    tmp_v[kk, :] = acc
                totA = jnp.zeros((16,), jnp.float32)
                totB = jnp.zeros((16,), jnp.float32)
                for j in range(16):
                    cj = jnp.full((16,), j, jnp.int32)
                    totA = totA + plsc.load_gather(tmp_v, [iota16, cj])
                    totB = totB + plsc.load_gather(tmp_v, [iota16 + 16, cj])
                n_out_v[ex, pl.ds(0, 16)] = totA
                n_out_v[ex, pl.ds(16, 16)] = totB

            @pl.when(p < NCHUNK // 2 - 1)
            def _():
                _neg_start(c + 2, b)
        return carry

    lax.fori_loop(0, NCHUNK // 2, neg_pair, 0)
    pltpu.sync_copy(n_out_v, n_hbm.at[pl.ds(b0, B_PER_W)])


def _gather_dots(input_s, missing_word, nwords, lookup_table):
    mesh = plsc.VectorSubcoreMesh(core_axis_name="c", subcore_axis_name="s")
    f = functools.partial(
        pl.kernel,
        out_type=(
            jax.ShapeDtypeStruct((BATCH,), jnp.float32),
            jax.ShapeDtypeStruct((BATCH, NPAD), jnp.float32),
        ),
        mesh=mesh,
        scratch_types=[
            pltpu.VMEM((B_PER_W * CTX,), jnp.int32),
            pltpu.VMEM((EX_PER_CHUNK * CTX, EMB), jnp.float32),
            pltpu.VMEM((EX_PER_CHUNK * CTX, EMB), jnp.float32),
            pltpu.VMEM((B_PER_W, EMB), jnp.float32),
            pltpu.VMEM((B_PER_W,), jnp.int32),
            pltpu.VMEM((B_PER_W, EMB), jnp.float32),
            pltpu.VMEM((SAMP_PER_W,), jnp.int32),
            pltpu.VMEM((NPAD, 16), jnp.float32),
            pltpu.VMEM((B_PER_W, NPAD), jnp.float32),
            pltpu.VMEM((B_PER_W,), jnp.float32),
            pltpu.SemaphoreType.DMA,
            pltpu.SemaphoreType.DMA,
            pltpu.SemaphoreType.DMA,
        ],
        compiler_params=pltpu.CompilerParams(needs_layout_passes=False),
    )(_gather_body)
    return f(input_s.reshape(BATCH * CTX), missing_word, nwords, lookup_table)


def _loss_body(is_ref, o_ref, n_ref, out_ref):
    is_blk = is_ref[...]
    o_raw = o_ref[...]
    n_raw = n_ref[...][:, :N_NEGS]
    cnt = jnp.sum((is_blk != PAD).astype(jnp.float32), axis=1)
    inv = 1.0 / jnp.maximum(cnt, 1.0)
    ol = jnp.log(jax.nn.sigmoid(o_raw * inv) + 1e-05)
    nl = jnp.log(jax.nn.sigmoid(-n_raw * inv[:, None]) + 1e-05)
    out_ref[...] = -(ol + jnp.mean(nl, axis=1))


def _loss(input_s, o_raw, n_raw):
    bb = 512
    grid = BATCH // bb
    return pl.pallas_call(
        _loss_body,
        grid=(grid,),
        in_specs=[
            pl.BlockSpec((bb, CTX), lambda i: (i, 0)),
            pl.BlockSpec((bb,), lambda i: (i,)),
            pl.BlockSpec((bb, NPAD), lambda i: (i, 0)),
        ],
        out_specs=pl.BlockSpec((bb,), lambda i: (i,)),
        out_shape=jax.ShapeDtypeStruct((BATCH,), jnp.float32),
    )(input_s, o_raw, n_raw)


def kernel(input_s, missing_word, lookup_table, weights):
    cdf_flat, u_flat = _cdf_and_uniforms(weights)
    nwords = _sample_negatives(cdf_flat, u_flat)
    o_raw, n_raw = _gather_dots(input_s, missing_word, nwords, lookup_table)
    return _loss(input_s, o_raw, n_raw)


# speedup vs baseline: 1.1204x; 1.1204x over previous
import functools

import jax
import jax.numpy as jnp
from jax import lax
from jax.experimental import pallas as pl
from jax.experimental.pallas import tpu as pltpu
from jax.experimental.pallas import tpu_sc as plsc

VOCAB = 100000
EMB = 128
BATCH = 4096
CTX = 50
N_NEGS = 20
PAD = 0
SEED = 42

NSAMP = BATCH * N_NEGS
ROWS = 784
VPAD = ROWS * 128
NW = 32
SAMP_PER_W = NSAMP // NW
B_PER_W = BATCH // NW
SEARCH_STEPS = 17
NPAD = 32
EX_PER_CHUNK = 4
NCHUNK = B_PER_W // EX_PER_CHUNK
JC = EMB // 16


def _cdf_rng_body(w_ref, cdf_ref, u_ref):
    w = w_ref[...]
    k = lax.broadcasted_iota(jnp.int32, (128, 128), 0)
    j = lax.broadcasted_iota(jnp.int32, (128, 128), 1)
    tri = (k <= j).astype(jnp.float32)
    rowcum = jnp.dot(w, tri, preferred_element_type=jnp.float32)
    totals = rowcum[:, 127:128]
    r1 = lax.broadcasted_iota(jnp.int32, (ROWS, ROWS), 0)
    r2 = lax.broadcasted_iota(jnp.int32, (ROWS, ROWS), 1)
    ls = (r2 < r1).astype(jnp.float32)
    prefix = jnp.dot(ls, totals, preferred_element_type=jnp.float32)
    cdf = rowcum + prefix
    cdf_ref[...] = cdf
    total = cdf[ROWS - 1:ROWS, 127:128]

    pltpu.prng_seed(SEED)
    bits = pltpu.prng_random_bits((NSAMP // 128, 128))
    bits = lax.bitcast_convert_type(bits, jnp.uint32)
    mant = lax.shift_right_logical(bits, jnp.uint32(9)) | jnp.uint32(0x3F800000)
    unif = lax.bitcast_convert_type(mant, jnp.float32) - 1.0
    u_ref[...] = unif * total


def _cdf_and_uniforms(weights):
    w2d = jnp.pad(weights, (0, VPAD - VOCAB)).reshape(ROWS, 128)
    cdf, u = pl.pallas_call(
        _cdf_rng_body,
        out_shape=(
            jax.ShapeDtypeStruct((ROWS, 128), jnp.float32),
            jax.ShapeDtypeStruct((NSAMP // 128, 128), jnp.float32),
        ),
    )(w2d)
    return cdf.reshape(VPAD), u.reshape(NSAMP)


def _sample_body(cdf_hbm, u_hbm, nw_hbm, cdf_v, u_v, out_v, sem):
    wid = lax.axis_index("s") * 2 + lax.axis_index("c")
    base = wid * SAMP_PER_W
    pltpu.async_copy(cdf_hbm, cdf_v, sem).wait()
    pltpu.sync_copy(u_hbm.at[pl.ds(base, SAMP_PER_W)], u_v)

    nlanes = 8

    def group(g, carry):
        ums = [u_v[pl.ds((nlanes * g + t) * 16, 16)] for t in range(nlanes)]
        lo0 = [jnp.zeros((16,), jnp.int32)] * nlanes
        hi0 = [jnp.full((16,), VOCAB - 1, jnp.int32)] * nlanes

        def step(_, lohi):
            los, his = lohi
            nlos, nhis = [], []
            for t in range(nlanes):
                mid = lax.shift_right_arithmetic(los[t] + his[t], 1)
                c = plsc.load_gather(cdf_v, [mid])
                gt = c > ums[t]
                nlos.append(jnp.where(gt, los[t], mid + 1))
                nhis.append(jnp.where(gt, mid, his[t]))
            return (nlos, nhis)

        _, his = lax.fori_loop(0, SEARCH_STEPS, step, (lo0, hi0))
        for t in range(nlanes):
            out_v[pl.ds((nlanes * g + t) * 16, 16)] = his[t]
        return carry

    lax.fori_loop(0, SAMP_PER_W // (16 * nlanes), group, 0)
    pltpu.sync_copy(out_v, nw_hbm.at[pl.ds(base, SAMP_PER_W)])


def _sample_negatives(cdf_flat, u_flat):
    mesh = plsc.VectorSubcoreMesh(core_axis_name="c", subcore_axis_name="s")
    f = functools.partial(
        pl.kernel,
        out_type=jax.ShapeDtypeStruct((NSAMP,), jnp.int32),
        mesh=mesh,
        scratch_types=[
            pltpu.VMEM((VPAD,), jnp.float32),
            pltpu.VMEM((SAMP_PER_W,), jnp.float32),
            pltpu.VMEM((SAMP_PER_W,), jnp.int32),
            pltpu.SemaphoreType.DMA,
        ],
        compiler_params=pltpu.CompilerParams(needs_layout_passes=False),
    )(_sample_body)
    return f(cdf_flat, u_flat)


def _gather_body(isf_hbm, mw_hbm, nw_hbm, tab_hbm, o_hbm, n_hbm,
                 sidx_v, rows0, rows1, embsum_v, midx_v, mrow_v,
                 negidx_v, tmp_v, n_out_v, o_out_v, sem0, sem1, sem2):
    sems = (sem0, sem1)
    rows_b = (rows0, rows1)
    wid = lax.axis_index("s") * 2 + lax.axis_index("c")
    b0 = wid * B_PER_W
    s0 = wid * SAMP_PER_W
    iota16 = lax.iota(jnp.int32, 16)

    pltpu.sync_copy(mw_hbm.at[pl.ds(b0, B_PER_W)], midx_v)
    pltpu.async_copy(tab_hbm.at[midx_v], mrow_v, sem2)

    pltpu.sync_copy(isf_hbm.at[pl.ds(b0 * CTX, B_PER_W * CTX)], sidx_v)

    def _ctx_start(c, b):
        pltpu.async_copy(
            tab_hbm.at[sidx_v.at[pl.ds(c * EX_PER_CHUNK * CTX,
                                       EX_PER_CHUNK * CTX)]],
            rows_b[b], sems[b])

    def _ctx_wait(c, b):
        pltpu.make_async_copy(
            tab_hbm.at[sidx_v.at[pl.ds(c * EX_PER_CHUNK * CTX,
                                       EX_PER_CHUNK * CTX)]],
            rows_b[b], sems[b]).wait()

    _ctx_start(0, 0)
    _ctx_start(1, 1)

    def ctx_pair(p, carry):
        for b in range(2):
            c = 2 * p + b
            _ctx_wait(c, b)
            for t in range(EX_PER_CHUNK):
                def rblk(rb, accs):
                    out = list(accs)
                    for rr in range(10):
                        r = t * CTX + rb * 10 + rr
                        for j in range(JC):
                            out[j] = out[j] + rows_b[b][r, pl.ds(j * 16, 16)]
                    return tuple(out)

                accs = lax.fori_loop(
                    0, CTX // 10, rblk,
                    tuple(jnp.zeros((16,), jnp.float32) for _ in range(JC)))
                ex = c * EX_PER_CHUNK + t
                for j in range(JC):
                    embsum_v[ex, pl.ds(j * 16, 16)] = accs[j]

            @pl.when(p < NCHUNK // 2 - 1)
            def _():
                _ctx_start(c + 2, b)
        return carry

    lax.fori_loop(0, NCHUNK // 2, ctx_pair, 0)

    pltpu.make_async_copy(tab_hbm.at[midx_v], mrow_v, sem2).wait()

    def ogroup(g, carry):
        for e in range(16):
            ex = g * 16 + e
            acc = jnp.zeros((16,), jnp.float32)
            for j in range(JC):
                acc = acc + (mrow_v[ex, pl.ds(j * 16, 16)] *
                             embsum_v[ex, pl.ds(j * 16, 16)])
            tmp_v[e, :] = acc
        tot = jnp.zeros((16,), jnp.float32)
        for j in range(16):
            tot = tot + plsc.load_gather(
                tmp_v, [iota16, jnp.full((16,), j, jnp.int32)])
        o_out_v[pl.ds(g * 16, 16)] = tot
        return carry

    lax.fori_loop(0, B_PER_W // 16, ogroup, 0)
    pltpu.sync_copy(o_out_v, o_hbm.at[pl.ds(b0, B_PER_W)])

    pltpu.sync_copy(nw_hbm.at[pl.ds(s0, SAMP_PER_W)], negidx_v)

    def _neg_start(c, b):
        pltpu.async_copy(
            tab_hbm.at[negidx_v.at[pl.ds(c * 80, 80)]], rows_b[b].at[:80],
            sems[b])

    def _neg_wait(c, b):
        pltpu.make_async_copy(
            tab_hbm.at[negidx_v.at[pl.ds(c * 80, 80)]], rows_b[b].at[:80],
            sems[b]).wait()

    _neg_start(0, 0)
    _neg_start(1, 1)

    def neg_pair(p, carry):
        for b in range(2):
            c = 2 * p + b
            _neg_wait(c, b)
            for t in range(EX_PER_CHUNK):
                ex = c * EX_PER_CHUNK + t
                embregs = [embsum_v[ex, pl.ds(j * 16, 16)] for j in range(JC)]

                def kdot(kk, carry2):
                    acc = jnp.zeros((16,), jnp.float32)
                    for j in range(JC):
                        acc = acc + (rows_b[b][t * N_NEGS + kk,
                                               pl.ds(j * 16, 16)] *
                                     embregs[j])
                    tmp_v[kk, :] = acc
                    return carry2

                lax.fori_loop(0, N_NEGS, kdot, 0)
                totA = jnp.zeros((16,), jnp.float32)
                totB = jnp.zeros((16,), jnp.float32)
                for j in range(16):
                    cj = jnp.full((16,), j, jnp.int32)
                    totA = totA + plsc.load_gather(tmp_v, [iota16, cj])
                    totB = totB + plsc.load_gather(tmp_v, [iota16 + 16, cj])
                n_out_v[ex, pl.ds(0, 16)] = totA
                n_out_v[ex, pl.ds(16, 16)] = totB

            @pl.when(p < NCHUNK // 2 - 1)
            def _():
                _neg_start(c + 2, b)
        return carry

    lax.fori_loop(0, NCHUNK // 2, neg_pair, 0)
    pltpu.sync_copy(n_out_v, n_hbm.at[pl.ds(b0, B_PER_W)])


def _gather_dots(input_s, missing_word, nwords, lookup_table):
    mesh = plsc.VectorSubcoreMesh(core_axis_name="c", subcore_axis_name="s")
    f = functools.partial(
        pl.kernel,
        out_type=(
            jax.ShapeDtypeStruct((BATCH,), jnp.float32),
            jax.ShapeDtypeStruct((BATCH, NPAD), jnp.float32),
        ),
        mesh=mesh,
        scratch_types=[
            pltpu.VMEM((B_PER_W * CTX,), jnp.int32),
            pltpu.VMEM((EX_PER_CHUNK * CTX, EMB), jnp.float32),
            pltpu.VMEM((EX_PER_CHUNK * CTX, EMB), jnp.float32),
            pltpu.VMEM((B_PER_W, EMB), jnp.float32),
            pltpu.VMEM((B_PER_W,), jnp.int32),
            pltpu.VMEM((B_PER_W, EMB), jnp.float32),
            pltpu.VMEM((SAMP_PER_W,), jnp.int32),
            pltpu.VMEM((NPAD, 16), jnp.float32),
            pltpu.VMEM((B_PER_W, NPAD), jnp.float32),
            pltpu.VMEM((B_PER_W,), jnp.float32),
            pltpu.SemaphoreType.DMA,
            pltpu.SemaphoreType.DMA,
            pltpu.SemaphoreType.DMA,
        ],
        compiler_params=pltpu.CompilerParams(needs_layout_passes=False),
    )(_gather_body)
    return f(input_s.reshape(BATCH * CTX), missing_word, nwords, lookup_table)


def _loss_body(is_ref, o_ref, n_ref, out_ref):
    is_blk = is_ref[...]
    o_raw = o_ref[...]
    n_raw = n_ref[...][:, :N_NEGS]
    cnt = jnp.sum((is_blk != PAD).astype(jnp.float32), axis=1)
    inv = 1.0 / jnp.maximum(cnt, 1.0)
    ol = jnp.log(jax.nn.sigmoid(o_raw * inv) + 1e-05)
    nl = jnp.log(jax.nn.sigmoid(-n_raw * inv[:, None]) + 1e-05)
    out_ref[...] = -(ol + jnp.mean(nl, axis=1))


def _loss(input_s, o_raw, n_raw):
    bb = 512
    grid = BATCH // bb
    return pl.pallas_call(
        _loss_body,
        grid=(grid,),
        in_specs=[
            pl.BlockSpec((bb, CTX), lambda i: (i, 0)),
            pl.BlockSpec((bb,), lambda i: (i,)),
            pl.BlockSpec((bb, NPAD), lambda i: (i, 0)),
        ],
        out_specs=pl.BlockSpec((bb,), lambda i: (i,)),
        out_shape=jax.ShapeDtypeStruct((BATCH,), jnp.float32),
    )(input_s, o_raw, n_raw)


def kernel(input_s, missing_word, lookup_table, weights):
    cdf_flat, u_flat = _cdf_and_uniforms(weights)
    nwords = _sample_negatives(cdf_flat, u_flat)
    o_raw, n_raw = _gather_dots(input_s, missing_word, nwords, lookup_table)
    return _loss(input_s, o_raw, n_raw)
